# Initial kernel scaffold; baseline (speedup 1.0000x reference)
import functools

import jax
import jax.numpy as jnp
from jax import lax
from jax.experimental import pallas as pl
from jax.experimental.pallas import tpu as pltpu
from jax.experimental.pallas import tpu_sc as plsc

N = 10000
E = 320000
D = 128
NCLS = 10

NC = 2
NS = 16
NW = NC * NS
EPW = E // NW
CHUNK = 80
NSTEP = EPW // CHUNK
RPT = N // NS

BR = 256
GRID = (N + BR - 1) // BR

_sc_mesh = plsc.VectorSubcoreMesh(
    core_axis_name="c", subcore_axis_name="s", num_cores=NC, num_subcores=NS
)


@functools.partial(
    pl.kernel,
    out_type=jax.ShapeDtypeStruct((NC, N, D), jnp.float32),
    mesh=_sc_mesh,
    scratch_types=[
        pltpu.VMEM((CHUNK,), jnp.int32),
        pltpu.VMEM((CHUNK,), jnp.int32),
        pltpu.VMEM((CHUNK, D), jnp.float32),
        pltpu.VMEM_SHARED((N, D), jnp.float32),
    ],
)
def _sc_propagate(t_hbm, src_hbm, dst_hbm, zeros_hbm, out_hbm,
                  srcv, dstv, rows, acc):
    c = lax.axis_index("c")
    s = lax.axis_index("s")
    wid = c * NS + s

    pltpu.sync_copy(zeros_hbm, acc.at[pl.ds(s * RPT, RPT)])
    plsc.subcore_barrier()

    def step(j, carry):
        base = wid * EPW + j * CHUNK
        pltpu.sync_copy(src_hbm.at[pl.ds(base, CHUNK)], srcv)
        pltpu.sync_copy(dst_hbm.at[pl.ds(base, CHUNK)], dstv)
        pltpu.sync_copy(t_hbm.at[srcv], rows)
        pltpu.sync_copy(rows, acc.at[dstv], add=True)
        return carry

    lax.fori_loop(0, NSTEP, step, 0)
    plsc.subcore_barrier()

    pltpu.sync_copy(acc.at[pl.ds(s * RPT, RPT)],
                    out_hbm.at[c, pl.ds(s * RPT, RPT)])


@functools.partial(
    pl.kernel,
    out_type=jax.ShapeDtypeStruct((NC, N, 16), jnp.float32),
    mesh=_sc_mesh,
    scratch_types=[
        pltpu.VMEM((CHUNK,), jnp.int32),
        pltpu.VMEM((CHUNK, 16), jnp.float32),
        pltpu.VMEM_SHARED((N, 16), jnp.float32),
    ],
)
def _sc_degree(dst_hbm, zeros_hbm, ones_hbm, out_hbm, dstv, onesv, acc):
    c = lax.axis_index("c")
    s = lax.axis_index("s")
    wid = c * NS + s

    pltpu.sync_copy(zeros_hbm, acc.at[pl.ds(s * RPT, RPT)])
    pltpu.sync_copy(ones_hbm, onesv)
    plsc.subcore_barrier()

    def step(j, carry):
        base = wid * EPW + j * CHUNK
        pltpu.sync_copy(dst_hbm.at[pl.ds(base, CHUNK)], dstv)
        pltpu.sync_copy(onesv, acc.at[dstv], add=True)
        return carry

    lax.fori_loop(0, NSTEP, step, 0)
    plsc.subcore_barrier()
    pltpu.sync_copy(acc.at[pl.ds(s * RPT, RPT)],
                    out_hbm.at[c, pl.ds(s * RPT, RPT)])


def _norm_prescale_body(degp_ref, h_ref, norm_ref, t0_ref):
    deg = jnp.sum(degp_ref[...], axis=1)
    nrm = lax.rsqrt(jnp.maximum(deg, 1.0))
    norm_ref[...] = nrm[:, None]
    t0_ref[...] = h_ref[...] * nrm[:, None]


def _norm_prescale(degp, h):
    return pl.pallas_call(
        _norm_prescale_body,
        grid=(GRID,),
        in_specs=[
            pl.BlockSpec((BR, 32), lambda i: (i, 0)),
            pl.BlockSpec((BR, D), lambda i: (i, 0)),
        ],
        out_specs=[
            pl.BlockSpec((BR, 1), lambda i: (i, 0)),
            pl.BlockSpec((BR, D), lambda i: (i, 0)),
        ],
        out_shape=[
            jax.ShapeDtypeStruct((N, 1), jnp.float32),
            jax.ShapeDtypeStruct((N, D), jnp.float32),
        ],
    )(degp, h)


def _midscale_body(p_ref, norm_ref, t_ref):
    nr = norm_ref[...]
    t_ref[...] = (p_ref[0] + p_ref[1]) * (nr * nr)


def _midscale(p, norm):
    return pl.pallas_call(
        _midscale_body,
        grid=(GRID,),
        in_specs=[
            pl.BlockSpec((NC, BR, D), lambda i: (0, i, 0)),
            pl.BlockSpec((BR, 1), lambda i: (i, 0)),
        ],
        out_specs=pl.BlockSpec((BR, D), lambda i: (i, 0)),
        out_shape=jax.ShapeDtypeStruct((N, D), jnp.float32),
    )(p, norm)


def _layer1_body(x0_ref, p1_ref, p2_ref, norm_ref, w_ref, b_ref,
                 h_ref, th_ref):
    nr = norm_ref[...]
    a1 = (p1_ref[0] + p1_ref[1]) * nr
    a2 = (p2_ref[0] + p2_ref[1]) * nr
    w = w_ref[...]
    acc = jnp.dot(x0_ref[...], w[:D], preferred_element_type=jnp.float32)
    acc += jnp.dot(a1, w[D:2 * D], preferred_element_type=jnp.float32)
    acc += jnp.dot(a2, w[2 * D:], preferred_element_type=jnp.float32)
    h = jnp.maximum(acc + b_ref[...], 0.0)
    h_ref[...] = h
    th_ref[...] = h * nr


def _layer1(x0, p1, p2, norm, W, b):
    return pl.pallas_call(
        _layer1_body,
        grid=(GRID,),
        in_specs=[
            pl.BlockSpec((BR, D), lambda i: (i, 0)),
            pl.BlockSpec((NC, BR, D), lambda i: (0, i, 0)),
            pl.BlockSpec((NC, BR, D), lambda i: (0, i, 0)),
            pl.BlockSpec((BR, 1), lambda i: (i, 0)),
            pl.BlockSpec((3 * D, D), lambda i: (0, 0)),
            pl.BlockSpec((1, D), lambda i: (0, 0)),
        ],
        out_specs=[
            pl.BlockSpec((BR, D), lambda i: (i, 0)),
            pl.BlockSpec((BR, D), lambda i: (i, 0)),
        ],
        out_shape=[
            jax.ShapeDtypeStruct((N, D), jnp.float32),
            jax.ShapeDtypeStruct((N, D), jnp.float32),
        ],
    )(x0, p1, p2, norm, W, b)


def _layer2_body(x0_ref, p1_ref, p2_ref, norm_ref, w_ref, b_ref,
                 wc_ref, bc_ref, out_ref, acc_ref):
    i = pl.program_id(0)
    nr = norm_ref[...]
    a1 = (p1_ref[0] + p1_ref[1]) * nr
    a2 = (p2_ref[0] + p2_ref[1]) * nr
    w = w_ref[...]
    acc = jnp.dot(x0_ref[...], w[:D], preferred_element_type=jnp.float32)
    acc += jnp.dot(a1, w[D:2 * D], preferred_element_type=jnp.float32)
    acc += jnp.dot(a2, w[2 * D:], preferred_element_type=jnp.float32)
    h = jnp.maximum(acc + b_ref[...], 0.0)
    gid = i * BR + lax.broadcasted_iota(jnp.int32, (BR, 1), 0)
    h = jnp.where(gid < N, h, -jnp.inf)
    m = jnp.max(h, axis=0, keepdims=True)

    @pl.when(i == 0)
    def _():
        acc_ref[...] = m

    @pl.when(i > 0)
    def _():
        acc_ref[...] = jnp.maximum(acc_ref[...], m)

    @pl.when(i == pl.num_programs(0) - 1)
    def _():
        out_ref[...] = (
            jnp.dot(acc_ref[...], wc_ref[...],
                    preferred_element_type=jnp.float32)
            + bc_ref[...]
        )


def _layer2(x0, p1, p2, norm, W, b, Wc, bc):
    return pl.pallas_call(
        _layer2_body,
        grid=(GRID,),
        in_specs=[
            pl.BlockSpec((BR, D), lambda i: (i, 0)),
            pl.BlockSpec((NC, BR, D), lambda i: (0, i, 0)),
            pl.BlockSpec((NC, BR, D), lambda i: (0, i, 0)),
            pl.BlockSpec((BR, 1), lambda i: (i, 0)),
            pl.BlockSpec((3 * D, D), lambda i: (0, 0)),
            pl.BlockSpec((1, D), lambda i: (0, 0)),
            pl.BlockSpec((D, NCLS), lambda i: (0, 0)),
            pl.BlockSpec((1, NCLS), lambda i: (0, 0)),
        ],
        out_specs=pl.BlockSpec((1, NCLS), lambda i: (0, 0)),
        out_shape=jax.ShapeDtypeStruct((1, NCLS), jnp.float32),
        scratch_shapes=[pltpu.VMEM((1, D), jnp.float32)],
    )(x0, p1, p2, norm, W, b, Wc, bc)


def kernel(h, edge_index, W1, b1, W2, b2, Wc, bc):
    src = edge_index[0]
    dst = edge_index[1]

    zeros_d = jnp.zeros((RPT, D), jnp.float32)
    zeros_16 = jnp.zeros((RPT, 16), jnp.float32)
    ones_rows = jnp.zeros((CHUNK, 16), jnp.float32).at[:, 0].set(1.0)

    degp = _sc_degree(dst, zeros_16, ones_rows)
    degp = jnp.transpose(degp, (1, 0, 2)).reshape(N, 32)
    norm, t0 = _norm_prescale(degp, h)

    b1r = b1.reshape(1, D)
    b2r = b2.reshape(1, D)
    bcr = bc.reshape(1, NCLS)

    p1 = _sc_propagate(t0, src, dst, zeros_d)
    t1 = _midscale(p1, norm)
    p2 = _sc_propagate(t1, src, dst, zeros_d)
    h1, th1 = _layer1(h, p1, p2, norm, W1, b1r)

    q1 = _sc_propagate(th1, src, dst, zeros_d)
    u1 = _midscale(q1, norm)
    q2 = _sc_propagate(u1, src, dst, zeros_d)
    return _layer2(h1, q1, q2, norm, W2, b2r, Wc, bcr)


# speedup vs baseline: 4.0283x; 4.0283x over previous
import functools

import jax
import jax.numpy as jnp
from jax import lax
from jax.experimental import pallas as pl
from jax.experimental.pallas import tpu as pltpu
from jax.experimental.pallas import tpu_sc as plsc

N = 10000
E = 320000
D = 128
NCLS = 10

NC = 2
NS = 16
NW = NC * NS
EPW = E // NW
CHUNK = 80
NSTEP = EPW // CHUNK
ZSTRIDE = 624
ZSPAN = 640

BR = 256
GRID = (N + BR - 1) // BR

@functools.cache
def _sc_kernels():
    mesh = plsc.VectorSubcoreMesh(
        core_axis_name="c", subcore_axis_name="s",
        num_cores=NC, num_subcores=NS,
    )

    @functools.partial(
        pl.kernel,
        out_type=jax.ShapeDtypeStruct((NC, N, D), jnp.float32),
        mesh=mesh,
        scratch_types=[
            pltpu.VMEM((CHUNK,), jnp.int32),
            pltpu.VMEM((CHUNK,), jnp.int32),
            pltpu.VMEM((CHUNK, D), jnp.float32),
            pltpu.VMEM_SHARED((N, D), jnp.float32),
        ],
    )
    def _sc_propagate(t_hbm, src_hbm, dst_hbm, zeros_hbm, out_hbm,
                      srcv, dstv, rows, acc):
        c = lax.axis_index("c")
        s = lax.axis_index("s")
        wid = c * NS + s

        pltpu.sync_copy(zeros_hbm, acc.at[pl.ds(s * ZSTRIDE, ZSPAN)])
        plsc.subcore_barrier()

        def step(j, carry):
            base = wid * EPW + j * CHUNK
            pltpu.sync_copy(src_hbm.at[pl.ds(base, CHUNK)], srcv)
            pltpu.sync_copy(dst_hbm.at[pl.ds(base, CHUNK)], dstv)
            pltpu.sync_copy(t_hbm.at[srcv], rows)
            pltpu.sync_copy(rows, acc.at[dstv], add=True)
            return carry

        lax.fori_loop(0, NSTEP, step, 0)
        plsc.subcore_barrier()

        pltpu.sync_copy(acc.at[pl.ds(s * ZSTRIDE, ZSPAN)],
                        out_hbm.at[c, pl.ds(s * ZSTRIDE, ZSPAN)])

    @functools.partial(
        pl.kernel,
        out_type=jax.ShapeDtypeStruct((NC, N, D), jnp.float32),
        mesh=mesh,
        scratch_types=[
            pltpu.VMEM((CHUNK,), jnp.int32),
            pltpu.VMEM((CHUNK, D), jnp.float32),
            pltpu.VMEM_SHARED((N, D), jnp.float32),
        ],
    )
    def _sc_degree(dst_hbm, zeros_hbm, ones_hbm, out_hbm, dstv, onesv, acc):
        c = lax.axis_index("c")
        s = lax.axis_index("s")
        wid = c * NS + s

        pltpu.sync_copy(zeros_hbm, acc.at[pl.ds(s * ZSTRIDE, ZSPAN)])
        pltpu.sync_copy(ones_hbm, onesv)
        plsc.subcore_barrier()

        def step(j, carry):
            base = wid * EPW + j * CHUNK
            pltpu.sync_copy(dst_hbm.at[pl.ds(base, CHUNK)], dstv)
            pltpu.sync_copy(onesv, acc.at[dstv], add=True)
            return carry

        lax.fori_loop(0, NSTEP, step, 0)
        plsc.subcore_barrier()
        pltpu.sync_copy(acc.at[pl.ds(s * ZSTRIDE, ZSPAN)],
                        out_hbm.at[c, pl.ds(s * ZSTRIDE, ZSPAN)])

    return _sc_propagate, _sc_degree


def _norm_prescale_body(degp_ref, h_ref, norm_ref, t0_ref):
    deg = degp_ref[0, :, 0:1] + degp_ref[1, :, 0:1]
    nrm = lax.rsqrt(jnp.maximum(deg, 1.0))
    norm_ref[...] = nrm
    t0_ref[...] = h_ref[...] * nrm


def _norm_prescale(degp, h):
    return pl.pallas_call(
        _norm_prescale_body,
        grid=(GRID,),
        in_specs=[
            pl.BlockSpec((NC, BR, D), lambda i: (0, i, 0)),
            pl.BlockSpec((BR, D), lambda i: (i, 0)),
        ],
        out_specs=[
            pl.BlockSpec((BR, 1), lambda i: (i, 0)),
            pl.BlockSpec((BR, D), lambda i: (i, 0)),
        ],
        out_shape=[
            jax.ShapeDtypeStruct((N, 1), jnp.float32),
            jax.ShapeDtypeStruct((N, D), jnp.float32),
        ],
    )(degp, h)


def _midscale_body(p_ref, norm_ref, t_ref):
    nr = norm_ref[...]
    t_ref[...] = (p_ref[0] + p_ref[1]) * (nr * nr)


def _midscale(p, norm):
    return pl.pallas_call(
        _midscale_body,
        grid=(GRID,),
        in_specs=[
            pl.BlockSpec((NC, BR, D), lambda i: (0, i, 0)),
            pl.BlockSpec((BR, 1), lambda i: (i, 0)),
        ],
        out_specs=pl.BlockSpec((BR, D), lambda i: (i, 0)),
        out_shape=jax.ShapeDtypeStruct((N, D), jnp.float32),
    )(p, norm)


def _layer1_body(x0_ref, p1_ref, p2_ref, norm_ref, w_ref, b_ref,
                 h_ref, th_ref):
    nr = norm_ref[...]
    a1 = (p1_ref[0] + p1_ref[1]) * nr
    a2 = (p2_ref[0] + p2_ref[1]) * nr
    w = w_ref[...]
    acc = jnp.dot(x0_ref[...], w[:D], preferred_element_type=jnp.float32)
    acc += jnp.dot(a1, w[D:2 * D], preferred_element_type=jnp.float32)
    acc += jnp.dot(a2, w[2 * D:], preferred_element_type=jnp.float32)
    h = jnp.maximum(acc + b_ref[...], 0.0)
    h_ref[...] = h
    th_ref[...] = h * nr


def _layer1(x0, p1, p2, norm, W, b):
    return pl.pallas_call(
        _layer1_body,
        grid=(GRID,),
        in_specs=[
            pl.BlockSpec((BR, D), lambda i: (i, 0)),
            pl.BlockSpec((NC, BR, D), lambda i: (0, i, 0)),
            pl.BlockSpec((NC, BR, D), lambda i: (0, i, 0)),
            pl.BlockSpec((BR, 1), lambda i: (i, 0)),
            pl.BlockSpec((3 * D, D), lambda i: (0, 0)),
            pl.BlockSpec((1, D), lambda i: (0, 0)),
        ],
        out_specs=[
            pl.BlockSpec((BR, D), lambda i: (i, 0)),
            pl.BlockSpec((BR, D), lambda i: (i, 0)),
        ],
        out_shape=[
            jax.ShapeDtypeStruct((N, D), jnp.float32),
            jax.ShapeDtypeStruct((N, D), jnp.float32),
        ],
    )(x0, p1, p2, norm, W, b)


def _layer2_body(x0_ref, p1_ref, p2_ref, norm_ref, w_ref, b_ref,
                 wc_ref, bc_ref, out_ref, acc_ref):
    i = pl.program_id(0)
    nr = norm_ref[...]
    a1 = (p1_ref[0] + p1_ref[1]) * nr
    a2 = (p2_ref[0] + p2_ref[1]) * nr
    w = w_ref[...]
    acc = jnp.dot(x0_ref[...], w[:D], preferred_element_type=jnp.float32)
    acc += jnp.dot(a1, w[D:2 * D], preferred_element_type=jnp.float32)
    acc += jnp.dot(a2, w[2 * D:], preferred_element_type=jnp.float32)
    h = jnp.maximum(acc + b_ref[...], 0.0)
    gid = i * BR + lax.broadcasted_iota(jnp.int32, (BR, 1), 0)
    h = jnp.where(gid < N, h, -jnp.inf)
    m = jnp.max(h, axis=0, keepdims=True)

    @pl.when(i == 0)
    def _():
        acc_ref[...] = m

    @pl.when(i > 0)
    def _():
        acc_ref[...] = jnp.maximum(acc_ref[...], m)

    @pl.when(i == pl.num_programs(0) - 1)
    def _():
        out_ref[...] = (
            jnp.dot(acc_ref[...], wc_ref[...],
                    preferred_element_type=jnp.float32)
            + bc_ref[...]
        )


def _layer2(x0, p1, p2, norm, W, b, Wc, bc):
    return pl.pallas_call(
        _layer2_body,
        grid=(GRID,),
        in_specs=[
            pl.BlockSpec((BR, D), lambda i: (i, 0)),
            pl.BlockSpec((NC, BR, D), lambda i: (0, i, 0)),
            pl.BlockSpec((NC, BR, D), lambda i: (0, i, 0)),
            pl.BlockSpec((BR, 1), lambda i: (i, 0)),
            pl.BlockSpec((3 * D, D), lambda i: (0, 0)),
            pl.BlockSpec((1, D), lambda i: (0, 0)),
            pl.BlockSpec((D, NCLS), lambda i: (0, 0)),
            pl.BlockSpec((1, NCLS), lambda i: (0, 0)),
        ],
        out_specs=pl.BlockSpec((1, NCLS), lambda i: (0, 0)),
        out_shape=jax.ShapeDtypeStruct((1, NCLS), jnp.float32),
        scratch_shapes=[pltpu.VMEM((1, D), jnp.float32)],
    )(x0, p1, p2, norm, W, b, Wc, bc)


def kernel(h, edge_index, W1, b1, W2, b2, Wc, bc):
    _sc_propagate, _sc_degree = _sc_kernels()
    src = edge_index[0]
    dst = edge_index[1]

    zeros_d = jnp.zeros((ZSPAN, D), jnp.float32)
    ones_rows = jnp.ones((CHUNK, D), jnp.float32)

    degp = _sc_degree(dst, zeros_d, ones_rows)
    norm, t0 = _norm_prescale(degp, h)

    b1r = b1.reshape(1, D)
    b2r = b2.reshape(1, D)
    bcr = bc.reshape(1, NCLS)

    p1 = _sc_propagate(t0, src, dst, zeros_d)
    t1 = _midscale(p1, norm)
    p2 = _sc_propagate(t1, src, dst, zeros_d)
    h1, th1 = _layer1(h, p1, p2, norm, W1, b1r)

    q1 = _sc_propagate(th1, src, dst, zeros_d)
    u1 = _midscale(q1, norm)
    q2 = _sc_propagate(u1, src, dst, zeros_d)
    return _layer2(h1, q1, q2, norm, W2, b2r, Wc, bcr)


# speedup vs baseline: 10.0220x; 2.4879x over previous
import functools

import jax
import jax.numpy as jnp
from jax import lax
from jax.experimental import pallas as pl
from jax.experimental.pallas import tpu as pltpu
from jax.experimental.pallas import tpu_sc as plsc

N = 10000
E = 320000
D = 128
NCLS = 10

NC = 2
NS = 16
NW = NC * NS
EPW = E // NW
CHUNK = 80
NSTEP = EPW // CHUNK
NBUF = 4
IDXR = 2 * NBUF
NITER = NSTEP // IDXR
NPAD = NITER * IDXR + IDXR
DEGG = 5
ZSTRIDE = 624
ZSPAN = 640

BR = 256
GRID = (N + BR - 1) // BR

@functools.cache
def _sc_kernels():
    mesh = plsc.VectorSubcoreMesh(
        core_axis_name="c", subcore_axis_name="s",
        num_cores=NC, num_subcores=NS,
    )

    @functools.partial(
        pl.kernel,
        out_type=jax.ShapeDtypeStruct((NC, N, D), jnp.float32),
        mesh=mesh,
        scratch_types=[
            pltpu.VMEM((IDXR, 2, CHUNK), jnp.int32),
            pltpu.VMEM((NBUF, CHUNK, D), jnp.float32),
            pltpu.VMEM_SHARED((N, D), jnp.float32),
        ] + [pltpu.SemaphoreType.DMA] * (IDXR + NBUF),
    )
    def _sc_propagate(t_hbm, e_hbm, zeros_hbm, out_hbm, idxb, rows, acc,
                      *sems):
        sem_i = sems[:IDXR]
        sem_r = sems[IDXR:]
        c = lax.axis_index("c")
        s = lax.axis_index("s")
        wid = c * NS + s

        pltpu.sync_copy(zeros_hbm, acc.at[pl.ds(s * ZSTRIDE, ZSPAN)])
        plsc.subcore_barrier()

        for t in range(IDXR):
            pltpu.async_copy(e_hbm.at[wid, t], idxb.at[t], sem_i[t])
        for t in range(NBUF):
            pltpu.make_async_copy(
                e_hbm.at[wid, t], idxb.at[t], sem_i[t]).wait()
            pltpu.async_copy(t_hbm.at[idxb.at[t, 0]], rows.at[t], sem_r[t])

        def group(i, carry):
            j0 = i * IDXR
            for t in range(IDXR):
                j = j0 + t
                rslot = t % NBUF
                pltpu.make_async_copy(
                    t_hbm.at[idxb.at[t, 0]], rows.at[rslot],
                    sem_r[rslot]).wait()
                pltpu.sync_copy(rows.at[rslot], acc.at[idxb.at[t, 1]],
                                add=True)
                pltpu.async_copy(e_hbm.at[wid, j + IDXR], idxb.at[t],
                                 sem_i[t])
                t2 = (t + NBUF) % IDXR
                pltpu.make_async_copy(
                    e_hbm.at[wid, t], idxb.at[t2], sem_i[t2]).wait()
                pltpu.async_copy(t_hbm.at[idxb.at[t2, 0]], rows.at[rslot],
                                 sem_r[rslot])
            return carry

        lax.fori_loop(0, NITER, group, 0)

        for t in range(NSTEP - NITER * IDXR):
            slot8 = (NITER * IDXR + t) % IDXR
            rslot = t % NBUF
            pltpu.make_async_copy(
                t_hbm.at[idxb.at[slot8, 0]], rows.at[rslot],
                sem_r[rslot]).wait()
            pltpu.sync_copy(rows.at[rslot], acc.at[idxb.at[slot8, 1]],
                            add=True)
            if t == 0:
                fs = (NITER * IDXR + NBUF) % IDXR
                pltpu.make_async_copy(
                    e_hbm.at[wid, t], idxb.at[fs], sem_i[fs]).wait()
                pltpu.async_copy(t_hbm.at[idxb.at[fs, 0]], rows.at[rslot],
                                 sem_r[rslot])
        for u in range(NSTEP, NPAD):
            slot8 = u % IDXR
            pltpu.make_async_copy(
                e_hbm.at[wid, 0], idxb.at[slot8], sem_i[slot8]).wait()

        plsc.subcore_barrier()
        pltpu.sync_copy(acc.at[pl.ds(s * ZSTRIDE, ZSPAN)],
                        out_hbm.at[c, pl.ds(s * ZSTRIDE, ZSPAN)])

    @functools.partial(
        pl.kernel,
        out_type=jax.ShapeDtypeStruct((NC, N, D), jnp.float32),
        mesh=mesh,
        scratch_types=[
            pltpu.VMEM((NSTEP, CHUNK), jnp.int32),
            pltpu.VMEM((CHUNK, D), jnp.float32),
            pltpu.VMEM_SHARED((N, D), jnp.float32),
        ],
    )
    def _sc_degree(dst_hbm, zeros_hbm, ones_hbm, out_hbm, dsti, onesv, acc):
        c = lax.axis_index("c")
        s = lax.axis_index("s")
        wid = c * NS + s

        pltpu.sync_copy(zeros_hbm, acc.at[pl.ds(s * ZSTRIDE, ZSPAN)])
        pltpu.sync_copy(ones_hbm, onesv)
        pltpu.sync_copy(dst_hbm.at[wid], dsti)
        plsc.subcore_barrier()

        def step(j, carry):
            pltpu.sync_copy(onesv, acc.at[dsti.at[j]], add=True)
            return carry

        lax.fori_loop(0, NSTEP, step, 0)
        plsc.subcore_barrier()
        pltpu.sync_copy(acc.at[pl.ds(s * ZSTRIDE, ZSPAN)],
                        out_hbm.at[c, pl.ds(s * ZSTRIDE, ZSPAN)])

    return _sc_propagate, _sc_degree


def _norm_prescale_body(degp_ref, h_ref, norm_ref, t0_ref):
    deg = degp_ref[0, :, 0:1] + degp_ref[1, :, 0:1]
    nrm = lax.rsqrt(jnp.maximum(deg, 1.0))
    norm_ref[...] = nrm
    t0_ref[...] = h_ref[...] * nrm


def _norm_prescale(degp, h):
    return pl.pallas_call(
        _norm_prescale_body,
        grid=(GRID,),
        in_specs=[
            pl.BlockSpec((NC, BR, D), lambda i: (0, i, 0)),
            pl.BlockSpec((BR, D), lambda i: (i, 0)),
        ],
        out_specs=[
            pl.BlockSpec((BR, 1), lambda i: (i, 0)),
            pl.BlockSpec((BR, D), lambda i: (i, 0)),
        ],
        out_shape=[
            jax.ShapeDtypeStruct((N, 1), jnp.float32),
            jax.ShapeDtypeStruct((N, D), jnp.float32),
        ],
    )(degp, h)


def _midscale_body(p_ref, norm_ref, t_ref):
    nr = norm_ref[...]
    t_ref[...] = (p_ref[0] + p_ref[1]) * (nr * nr)


def _midscale(p, norm):
    return pl.pallas_call(
        _midscale_body,
        grid=(GRID,),
        in_specs=[
            pl.BlockSpec((NC, BR, D), lambda i: (0, i, 0)),
            pl.BlockSpec((BR, 1), lambda i: (i, 0)),
        ],
        out_specs=pl.BlockSpec((BR, D), lambda i: (i, 0)),
        out_shape=jax.ShapeDtypeStruct((N, D), jnp.float32),
    )(p, norm)


def _layer1_body(x0_ref, p1_ref, p2_ref, norm_ref, w_ref, b_ref,
                 h_ref, th_ref):
    nr = norm_ref[...]
    a1 = (p1_ref[0] + p1_ref[1]) * nr
    a2 = (p2_ref[0] + p2_ref[1]) * nr
    w = w_ref[...]
    acc = jnp.dot(x0_ref[...], w[:D], preferred_element_type=jnp.float32)
    acc += jnp.dot(a1, w[D:2 * D], preferred_element_type=jnp.float32)
    acc += jnp.dot(a2, w[2 * D:], preferred_element_type=jnp.float32)
    h = jnp.maximum(acc + b_ref[...], 0.0)
    h_ref[...] = h
    th_ref[...] = h * nr


def _layer1(x0, p1, p2, norm, W, b):
    return pl.pallas_call(
        _layer1_body,
        grid=(GRID,),
        in_specs=[
            pl.BlockSpec((BR, D), lambda i: (i, 0)),
            pl.BlockSpec((NC, BR, D), lambda i: (0, i, 0)),
            pl.BlockSpec((NC, BR, D), lambda i: (0, i, 0)),
            pl.BlockSpec((BR, 1), lambda i: (i, 0)),
            pl.BlockSpec((3 * D, D), lambda i: (0, 0)),
            pl.BlockSpec((1, D), lambda i: (0, 0)),
        ],
        out_specs=[
            pl.BlockSpec((BR, D), lambda i: (i, 0)),
            pl.BlockSpec((BR, D), lambda i: (i, 0)),
        ],
        out_shape=[
            jax.ShapeDtypeStruct((N, D), jnp.float32),
            jax.ShapeDtypeStruct((N, D), jnp.float32),
        ],
    )(x0, p1, p2, norm, W, b)


def _layer2_body(x0_ref, p1_ref, p2_ref, norm_ref, w_ref, b_ref,
                 wc_ref, bc_ref, out_ref, acc_ref):
    i = pl.program_id(0)
    nr = norm_ref[...]
    a1 = (p1_ref[0] + p1_ref[1]) * nr
    a2 = (p2_ref[0] + p2_ref[1]) * nr
    w = w_ref[...]
    acc = jnp.dot(x0_ref[...], w[:D], preferred_element_type=jnp.float32)
    acc += jnp.dot(a1, w[D:2 * D], preferred_element_type=jnp.float32)
    acc += jnp.dot(a2, w[2 * D:], preferred_element_type=jnp.float32)
    h = jnp.maximum(acc + b_ref[...], 0.0)
    gid = i * BR + lax.broadcasted_iota(jnp.int32, (BR, 1), 0)
    h = jnp.where(gid < N, h, -jnp.inf)
    m = jnp.max(h, axis=0, keepdims=True)

    @pl.when(i == 0)
    def _():
        acc_ref[...] = m

    @pl.when(i > 0)
    def _():
        acc_ref[...] = jnp.maximum(acc_ref[...], m)

    @pl.when(i == pl.num_programs(0) - 1)
    def _():
        out_ref[...] = (
            jnp.dot(acc_ref[...], wc_ref[...],
                    preferred_element_type=jnp.float32)
            + bc_ref[...]
        )


def _layer2(x0, p1, p2, norm, W, b, Wc, bc):
    return pl.pallas_call(
        _layer2_body,
        grid=(GRID,),
        in_specs=[
            pl.BlockSpec((BR, D), lambda i: (i, 0)),
            pl.BlockSpec((NC, BR, D), lambda i: (0, i, 0)),
            pl.BlockSpec((NC, BR, D), lambda i: (0, i, 0)),
            pl.BlockSpec((BR, 1), lambda i: (i, 0)),
            pl.BlockSpec((3 * D, D), lambda i: (0, 0)),
            pl.BlockSpec((1, D), lambda i: (0, 0)),
            pl.BlockSpec((D, NCLS), lambda i: (0, 0)),
            pl.BlockSpec((1, NCLS), lambda i: (0, 0)),
        ],
        out_specs=pl.BlockSpec((1, NCLS), lambda i: (0, 0)),
        out_shape=jax.ShapeDtypeStruct((1, NCLS), jnp.float32),
        scratch_shapes=[pltpu.VMEM((1, D), jnp.float32)],
    )(x0, p1, p2, norm, W, b, Wc, bc)


def kernel(h, edge_index, W1, b1, W2, b2, Wc, bc):
    _sc_propagate, _sc_degree = _sc_kernels()
    e3 = jnp.transpose(edge_index.reshape(2, NW, NSTEP, CHUNK), (1, 2, 0, 3))
    epad = jnp.concatenate(
        [e3, jnp.zeros((NW, NPAD - NSTEP, 2, CHUNK), jnp.int32)], axis=1)
    dst = edge_index[1].reshape(NW, NSTEP, CHUNK)

    zeros_d = jnp.zeros((ZSPAN, D), jnp.float32)
    ones_rows = jnp.ones((CHUNK, D), jnp.float32)

    degp = _sc_degree(dst, zeros_d, ones_rows)
    norm, t0 = _norm_prescale(degp, h)

    b1r = b1.reshape(1, D)
    b2r = b2.reshape(1, D)
    bcr = bc.reshape(1, NCLS)

    p1 = _sc_propagate(t0, epad, zeros_d)
    t1 = _midscale(p1, norm)
    p2 = _sc_propagate(t1, epad, zeros_d)
    h1, th1 = _layer1(h, p1, p2, norm, W1, b1r)

    q1 = _sc_propagate(th1, epad, zeros_d)
    u1 = _midscale(q1, norm)
    q2 = _sc_propagate(u1, epad, zeros_d)
    return _layer2(h1, q1, q2, norm, W2, b2r, Wc, bcr)


# speedup vs baseline: 11.7117x; 1.1686x over previous
import functools

import jax
import jax.numpy as jnp
from jax import lax
from jax.experimental import pallas as pl
from jax.experimental.pallas import tpu as pltpu
from jax.experimental.pallas import tpu_sc as plsc

N = 10000
E = 320000
D = 128
NCLS = 10

NC = 2
NS = 16
NW = NC * NS
EPW = E // NW
CHUNK = 80
NSTEP = EPW // CHUNK
NBUF = 4
IDXR = 2 * NBUF
NITER = NSTEP // IDXR
ZSTRIDE = 624
ZSPAN = 640

BR = 2048
GRID = (N + BR - 1) // BR

@functools.cache
def _sc_kernels():
    mesh = plsc.VectorSubcoreMesh(
        core_axis_name="c", subcore_axis_name="s",
        num_cores=NC, num_subcores=NS,
    )

    @functools.partial(
        pl.kernel,
        out_type=jax.ShapeDtypeStruct((NC, N, D), jnp.float32),
        mesh=mesh,
        scratch_types=[
            pltpu.VMEM((IDXR, 1, CHUNK), jnp.int32),
            pltpu.VMEM((IDXR, 1, CHUNK), jnp.int32),
            pltpu.VMEM((NBUF, CHUNK, D), jnp.float32),
            pltpu.VMEM_SHARED((N, D), jnp.float32),
        ] + [pltpu.SemaphoreType.DMA] * (IDXR + NBUF),
    )
    def _sc_propagate(t_hbm, e_hbm, zeros_hbm, out_hbm, srcb, dstb, rows,
                      acc, *sems):
        sem_i = sems[:IDXR]
        sem_r = sems[IDXR:]
        c = lax.axis_index("c")
        s = lax.axis_index("s")
        wid = c * NS + s

        def issue_idx(jc, slot):
            pltpu.async_copy(e_hbm.at[0, wid, jc], srcb.at[slot],
                             sem_i[slot])
            pltpu.async_copy(e_hbm.at[1, wid, jc], dstb.at[slot],
                             sem_i[slot])

        def wait_idx(slot):
            pltpu.make_async_copy(
                e_hbm.at[0, wid, 0], srcb.at[slot], sem_i[slot]).wait()
            pltpu.make_async_copy(
                e_hbm.at[1, wid, 0], dstb.at[slot], sem_i[slot]).wait()

        pltpu.sync_copy(zeros_hbm, acc.at[pl.ds(s * ZSTRIDE, ZSPAN)])
        plsc.subcore_barrier()

        for t in range(IDXR):
            issue_idx(t, t)
        for t in range(NBUF):
            wait_idx(t)
            pltpu.async_copy(t_hbm.at[srcb.at[t, 0]], rows.at[t], sem_r[t])

        def group(i, carry):
            j0 = i * IDXR
            for t in range(IDXR):
                j = j0 + t
                rslot = t % NBUF
                pltpu.make_async_copy(
                    t_hbm.at[srcb.at[t, 0]], rows.at[rslot],
                    sem_r[rslot]).wait()
                pltpu.sync_copy(rows.at[rslot], acc.at[dstb.at[t, 0]],
                                add=True)
                issue_idx(jnp.minimum(j + IDXR, NSTEP - 1), t)
                t2 = (t + NBUF) % IDXR
                wait_idx(t2)
                pltpu.async_copy(t_hbm.at[srcb.at[t2, 0]], rows.at[rslot],
                                 sem_r[rslot])
            return carry

        lax.fori_loop(0, NITER, group, 0)

        for t in range(NSTEP - NITER * IDXR):
            slot8 = (NITER * IDXR + t) % IDXR
            rslot = t % NBUF
            pltpu.make_async_copy(
                t_hbm.at[srcb.at[slot8, 0]], rows.at[rslot],
                sem_r[rslot]).wait()
            pltpu.sync_copy(rows.at[rslot], acc.at[dstb.at[slot8, 0]],
                            add=True)
            if t == 0:
                fs = (NITER * IDXR + NBUF) % IDXR
                wait_idx(fs)
                pltpu.async_copy(t_hbm.at[srcb.at[fs, 0]], rows.at[rslot],
                                 sem_r[rslot])
        for u in range(NSTEP, NITER * IDXR + IDXR):
            wait_idx(u % IDXR)

        plsc.subcore_barrier()
        pltpu.sync_copy(acc.at[pl.ds(s * ZSTRIDE, ZSPAN)],
                        out_hbm.at[c, pl.ds(s * ZSTRIDE, ZSPAN)])

    @functools.partial(
        pl.kernel,
        out_type=jax.ShapeDtypeStruct((NC, N, D), jnp.float32),
        mesh=mesh,
        scratch_types=[
            pltpu.VMEM((NSTEP, 1, CHUNK), jnp.int32),
            pltpu.VMEM((CHUNK, D), jnp.float32),
            pltpu.VMEM_SHARED((N, D), jnp.float32),
        ],
    )
    def _sc_degree(e_hbm, zeros_hbm, ones_hbm, out_hbm, dsti, onesv, acc):
        c = lax.axis_index("c")
        s = lax.axis_index("s")
        wid = c * NS + s

        pltpu.sync_copy(zeros_hbm, acc.at[pl.ds(s * ZSTRIDE, ZSPAN)])
        pltpu.sync_copy(ones_hbm, onesv)
        pltpu.sync_copy(e_hbm.at[1, wid], dsti)
        plsc.subcore_barrier()

        def step(j, carry):
            pltpu.sync_copy(onesv, acc.at[dsti.at[j, 0]], add=True)
            return carry

        lax.fori_loop(0, NSTEP, step, 0)
        plsc.subcore_barrier()
        pltpu.sync_copy(acc.at[pl.ds(s * ZSTRIDE, ZSPAN)],
                        out_hbm.at[c, pl.ds(s * ZSTRIDE, ZSPAN)])

    return _sc_propagate, _sc_degree


def _norm_prescale_body(degp_ref, h_ref, norm_ref, rnorm_ref, t0_ref):
    deg = jnp.maximum(degp_ref[0, :, 0:1] + degp_ref[1, :, 0:1], 1.0)
    nrm = lax.rsqrt(deg)
    norm_ref[...] = nrm
    rnorm_ref[...] = jnp.sqrt(deg)
    t0_ref[...] = h_ref[...] * nrm


def _norm_prescale(degp, h):
    return pl.pallas_call(
        _norm_prescale_body,
        grid=(GRID,),
        in_specs=[
            pl.BlockSpec((NC, BR, D), lambda i: (0, i, 0)),
            pl.BlockSpec((BR, D), lambda i: (i, 0)),
        ],
        out_specs=[
            pl.BlockSpec((BR, 1), lambda i: (i, 0)),
            pl.BlockSpec((BR, 1), lambda i: (i, 0)),
            pl.BlockSpec((BR, D), lambda i: (i, 0)),
        ],
        out_shape=[
            jax.ShapeDtypeStruct((N, 1), jnp.float32),
            jax.ShapeDtypeStruct((N, 1), jnp.float32),
            jax.ShapeDtypeStruct((N, D), jnp.float32),
        ],
    )(degp, h)


def _midscale_body(p_ref, norm_ref, t_ref):
    nr = norm_ref[...]
    t_ref[...] = (p_ref[0] + p_ref[1]) * (nr * nr)


def _midscale(p, norm):
    return pl.pallas_call(
        _midscale_body,
        grid=(GRID,),
        in_specs=[
            pl.BlockSpec((NC, BR, D), lambda i: (0, i, 0)),
            pl.BlockSpec((BR, 1), lambda i: (i, 0)),
        ],
        out_specs=pl.BlockSpec((BR, D), lambda i: (i, 0)),
        out_shape=jax.ShapeDtypeStruct((N, D), jnp.float32),
    )(p, norm)


def _layer1_body(x0_ref, p1_ref, p2_ref, norm_ref, w_ref, b_ref,
                 th_ref):
    nr = norm_ref[...]
    a1 = (p1_ref[0] + p1_ref[1]) * nr
    a2 = (p2_ref[0] + p2_ref[1]) * nr
    w = w_ref[...]
    acc = jnp.dot(x0_ref[...], w[:D], preferred_element_type=jnp.float32)
    acc += jnp.dot(a1, w[D:2 * D], preferred_element_type=jnp.float32)
    acc += jnp.dot(a2, w[2 * D:], preferred_element_type=jnp.float32)
    h = jnp.maximum(acc + b_ref[...], 0.0)
    th_ref[...] = h * nr


def _layer1(x0, p1, p2, norm, W, b):
    return pl.pallas_call(
        _layer1_body,
        grid=(GRID,),
        in_specs=[
            pl.BlockSpec((BR, D), lambda i: (i, 0)),
            pl.BlockSpec((NC, BR, D), lambda i: (0, i, 0)),
            pl.BlockSpec((NC, BR, D), lambda i: (0, i, 0)),
            pl.BlockSpec((BR, 1), lambda i: (i, 0)),
            pl.BlockSpec((3 * D, D), lambda i: (0, 0)),
            pl.BlockSpec((1, D), lambda i: (0, 0)),
        ],
        out_specs=pl.BlockSpec((BR, D), lambda i: (i, 0)),
        out_shape=jax.ShapeDtypeStruct((N, D), jnp.float32),
    )(x0, p1, p2, norm, W, b)


def _layer2_body(th_ref, p1_ref, p2_ref, norm_ref, rnorm_ref, w_ref, b_ref,
                 wc_ref, bc_ref, out_ref, acc_ref):
    i = pl.program_id(0)
    nr = norm_ref[...]
    a1 = (p1_ref[0] + p1_ref[1]) * nr
    a2 = (p2_ref[0] + p2_ref[1]) * nr
    w = w_ref[...]
    x0 = th_ref[...] * rnorm_ref[...]
    acc = jnp.dot(x0, w[:D], preferred_element_type=jnp.float32)
    acc += jnp.dot(a1, w[D:2 * D], preferred_element_type=jnp.float32)
    acc += jnp.dot(a2, w[2 * D:], preferred_element_type=jnp.float32)
    h = jnp.maximum(acc + b_ref[...], 0.0)
    gid = i * BR + lax.broadcasted_iota(jnp.int32, (BR, 1), 0)
    h = jnp.where(gid < N, h, -jnp.inf)
    m = jnp.max(h, axis=0, keepdims=True)

    @pl.when(i == 0)
    def _():
        acc_ref[...] = m

    @pl.when(i > 0)
    def _():
        acc_ref[...] = jnp.maximum(acc_ref[...], m)

    @pl.when(i == pl.num_programs(0) - 1)
    def _():
        out_ref[...] = (
            jnp.dot(acc_ref[...], wc_ref[...],
                    preferred_element_type=jnp.float32)
            + bc_ref[...]
        )


def _layer2(x0, p1, p2, norm, rnorm, W, b, Wc, bc):
    return pl.pallas_call(
        _layer2_body,
        grid=(GRID,),
        in_specs=[
            pl.BlockSpec((BR, D), lambda i: (i, 0)),
            pl.BlockSpec((NC, BR, D), lambda i: (0, i, 0)),
            pl.BlockSpec((NC, BR, D), lambda i: (0, i, 0)),
            pl.BlockSpec((BR, 1), lambda i: (i, 0)),
            pl.BlockSpec((BR, 1), lambda i: (i, 0)),
            pl.BlockSpec((3 * D, D), lambda i: (0, 0)),
            pl.BlockSpec((1, D), lambda i: (0, 0)),
            pl.BlockSpec((D, NCLS), lambda i: (0, 0)),
            pl.BlockSpec((1, NCLS), lambda i: (0, 0)),
        ],
        out_specs=pl.BlockSpec((1, NCLS), lambda i: (0, 0)),
        out_shape=jax.ShapeDtypeStruct((1, NCLS), jnp.float32),
        scratch_shapes=[pltpu.VMEM((1, D), jnp.float32)],
    )(x0, p1, p2, norm, rnorm, W, b, Wc, bc)


def kernel(h, edge_index, W1, b1, W2, b2, Wc, bc):
    _sc_propagate, _sc_degree = _sc_kernels()
    e5 = edge_index.reshape(2, NW, NSTEP, 1, CHUNK)

    zeros_d = jnp.zeros((ZSPAN, D), jnp.float32)
    ones_rows = jnp.ones((CHUNK, D), jnp.float32)

    degp = _sc_degree(e5, zeros_d, ones_rows)
    norm, rnorm, t0 = _norm_prescale(degp, h)

    b1r = b1.reshape(1, D)
    b2r = b2.reshape(1, D)
    bcr = bc.reshape(1, NCLS)

    p1 = _sc_propagate(t0, e5, zeros_d)
    t1 = _midscale(p1, norm)
    p2 = _sc_propagate(t1, e5, zeros_d)
    th1 = _layer1(h, p1, p2, norm, W1, b1r)

    q1 = _sc_propagate(th1, e5, zeros_d)
    u1 = _midscale(q1, norm)
    q2 = _sc_propagate(u1, e5, zeros_d)
    return _layer2(th1, q1, q2, norm, rnorm, W2, b2r, Wc, bcr)
